# baseline (device time: 13533 ns/iter reference)
import jax
import jax.numpy as jnp
from jax import lax
from jax.experimental import pallas as pl
from jax.experimental.pallas import tpu as pltpu

N_DEV = 4
B, SQ, SKV = 2, 128, 128
HQ_LOCAL, DH = 4, 64
D_MODEL = 512
CHUNK = HQ_LOCAL * DH
ROWS = B * SQ


def kernel(x, Wq, K_ext, V_ext, Wo):
    my_pos = lax.axis_index("i")
    Wq_loc = lax.dynamic_slice_in_dim(Wq, my_pos * CHUNK, CHUNK, axis=1)
    Wq_loc = Wq_loc.astype(jnp.bfloat16)
    x_bf = x.astype(jnp.bfloat16)
    KV = jnp.stack([
        jnp.transpose(K_ext, (0, 2, 3, 1)),
        jnp.transpose(V_ext, (0, 2, 3, 1)),
    ]).astype(jnp.bfloat16)

    def body(x_ref, wq_ref, kv_ref, wo_ref, out_ref,
             comm_ref, send_sems, recv_sems):
        me = lax.axis_index("i")

        barrier_sem = pltpu.get_barrier_semaphore()
        for d in range(1, N_DEV):
            pl.semaphore_signal(
                barrier_sem, inc=1,
                device_id=((me + d) % N_DEV,),
                device_id_type=pl.DeviceIdType.MESH,
            )

        x2d = x_ref[...].reshape(ROWS, D_MODEL)
        q_all = jnp.dot(x2d, wq_ref[...],
                        preferred_element_type=jnp.float32)
        q_all = q_all.astype(jnp.bfloat16)

        rdmas = {}
        BLK = SQ // 2
        for b in range(B):
            for h in range(HQ_LOCAL):
                qh = q_all[b * SQ:(b + 1) * SQ, h * DH:(h + 1) * DH]
                khT = kv_ref[0, b, h]
                vhT = kv_ref[1, b, h]
                s0 = jnp.dot(qh[:BLK], khT[:, :BLK],
                             preferred_element_type=jnp.float32) * 0.125
                m0 = jnp.max(s0, axis=1, keepdims=True)
                w0 = jnp.exp(s0 - m0)
                w0 = (w0 / jnp.sum(w0, axis=1, keepdims=True)).astype(
                    jnp.bfloat16)
                ctx0 = lax.dot_general(
                    w0, vhT[:, :BLK], (((1,), (1,)), ((), ())),
                    preferred_element_type=jnp.float32,
                )
                s1 = jnp.dot(qh[BLK:], khT,
                             preferred_element_type=jnp.float32) * 0.125
                m1 = jnp.max(s1, axis=1, keepdims=True)
                w1 = jnp.exp(s1 - m1)
                w1 = (w1 / jnp.sum(w1, axis=1, keepdims=True)).astype(
                    jnp.bfloat16)
                ctx1 = lax.dot_general(
                    w1, vhT, (((1,), (1,)), ((), ())),
                    preferred_element_type=jnp.float32,
                )
                cols = slice(h * DH, (h + 1) * DH)
                comm_ref[0, b * SQ:b * SQ + BLK, cols] = (
                    ctx0.astype(jnp.bfloat16))
                comm_ref[0, b * SQ + BLK:(b + 1) * SQ, cols] = (
                    ctx1.astype(jnp.bfloat16))

            if b == 0:
                pl.semaphore_wait(barrier_sem, N_DEV - 1)
            rows = pl.ds(b * SQ, SQ)
            for d in (2, 1, 3):
                rdmas[(d, b)] = pltpu.make_async_remote_copy(
                    src_ref=comm_ref.at[0, rows],
                    dst_ref=comm_ref.at[d, rows],
                    send_sem=send_sems.at[d - 1, b],
                    recv_sem=recv_sems.at[d - 1, b],
                    device_id=((me + d) % N_DEV,),
                    device_id_type=pl.DeviceIdType.MESH,
                )
                rdmas[(d, b)].start()

        wo_mine = wo_ref[pl.ds(me * CHUNK, CHUNK), :].astype(jnp.bfloat16)
        out2d = jnp.dot(comm_ref[0], wo_mine,
                        preferred_element_type=jnp.float32)

        for d in (1, 3, 2):
            origin = (me - d) % N_DEV
            wo_blk = wo_ref[pl.ds(origin * CHUNK, CHUNK), :].astype(
                jnp.bfloat16)
            for b in range(B):
                rdmas[(d, b)].wait_recv()
            out2d = out2d + jnp.dot(comm_ref[d], wo_blk,
                                    preferred_element_type=jnp.float32)
        for d in (1, 2, 3):
            for b in range(B):
                rdmas[(d, b)].wait_send()

        out_ref[...] = out2d.astype(jnp.bfloat16).reshape(B, SQ, D_MODEL)

    return pl.pallas_call(
        body,
        out_shape=jax.ShapeDtypeStruct((B, SQ, D_MODEL), jnp.bfloat16),
        in_specs=[pl.BlockSpec(memory_space=pltpu.VMEM)] * 4,
        out_specs=pl.BlockSpec(memory_space=pltpu.VMEM),
        scratch_shapes=[
            pltpu.VMEM((N_DEV, ROWS, CHUNK), jnp.bfloat16),
            pltpu.SemaphoreType.DMA((N_DEV - 1, B)),
            pltpu.SemaphoreType.DMA((N_DEV - 1, B)),
        ],
        compiler_params=pltpu.CompilerParams(collective_id=0),
    )(x_bf, Wq_loc, KV, Wo)


# device time: 12087 ns/iter; 1.1196x vs baseline; 1.1196x over previous
import jax
import jax.numpy as jnp
from jax import lax
from jax.experimental import pallas as pl
from jax.experimental.pallas import tpu as pltpu

N_DEV = 4
B, SQ, SKV = 2, 128, 128
HQ_LOCAL, DH = 4, 64
D_MODEL = 512
CHUNK = HQ_LOCAL * DH
ROWS = B * SQ


def kernel(x, Wq, K_ext, V_ext, Wo):
    my_pos = lax.axis_index("i")
    Wq_loc = lax.dynamic_slice_in_dim(Wq, my_pos * CHUNK, CHUNK, axis=1)
    Wq_loc = Wq_loc.astype(jnp.bfloat16)
    x_bf = x.astype(jnp.bfloat16)
    KV = jnp.stack([
        jnp.transpose(K_ext, (0, 2, 3, 1)),
        jnp.transpose(V_ext, (0, 2, 3, 1)),
    ]).astype(jnp.bfloat16)

    def body(x_ref, wq_ref, kv_ref, wo_ref, out_ref,
             comm_ref, send_sems, recv_sems):
        me = lax.axis_index("i")

        barrier_sem = pltpu.get_barrier_semaphore()
        for d in range(1, N_DEV):
            pl.semaphore_signal(
                barrier_sem, inc=1,
                device_id=((me + d) % N_DEV,),
                device_id_type=pl.DeviceIdType.MESH,
            )

        x2d = x_ref[...].reshape(ROWS, D_MODEL)
        q_all = jnp.dot(x2d, wq_ref[...],
                        preferred_element_type=jnp.float32)
        q_all = q_all.astype(jnp.bfloat16)

        rdmas = {}
        ii = lax.broadcasted_iota(jnp.int32, (SQ, SKV), 0)
        jj = lax.broadcasted_iota(jnp.int32, (SQ, SKV), 1)
        mask = (jj // 64) <= (ii // 64)

        for b in range(B):
            for h in range(HQ_LOCAL):
                qh = q_all[b * SQ:(b + 1) * SQ, h * DH:(h + 1) * DH]
                khT = kv_ref[0, b, h]
                s = jnp.dot(qh, khT,
                            preferred_element_type=jnp.float32) * 0.125
                s = jnp.where(mask, s, -1e9)
                w = jnp.exp(s)
                w = (w / jnp.sum(w, axis=1, keepdims=True)).astype(jnp.bfloat16)
                ctx_h = lax.dot_general(
                    w, kv_ref[1, b, h], (((1,), (1,)), ((), ())),
                    preferred_element_type=jnp.float32,
                )
                comm_ref[0, b * SQ:(b + 1) * SQ, h * DH:(h + 1) * DH] = (
                    ctx_h.astype(jnp.bfloat16))

            if b == 0:
                pl.semaphore_wait(barrier_sem, N_DEV - 1)
            rows = pl.ds(b * SQ, SQ)
            for d in (2, 1, 3):
                rdmas[(d, b)] = pltpu.make_async_remote_copy(
                    src_ref=comm_ref.at[0, rows],
                    dst_ref=comm_ref.at[d, rows],
                    send_sem=send_sems.at[d - 1, b],
                    recv_sem=recv_sems.at[d - 1, b],
                    device_id=((me + d) % N_DEV,),
                    device_id_type=pl.DeviceIdType.MESH,
                )
                rdmas[(d, b)].start()

        wo_mine = wo_ref[pl.ds(me * CHUNK, CHUNK), :].astype(jnp.bfloat16)
        out2d = jnp.dot(comm_ref[0], wo_mine,
                        preferred_element_type=jnp.float32)

        for d in (1, 3, 2):
            origin = (me - d) % N_DEV
            wo_blk = wo_ref[pl.ds(origin * CHUNK, CHUNK), :].astype(
                jnp.bfloat16)
            for b in range(B):
                rdmas[(d, b)].wait_recv()
            out2d = out2d + jnp.dot(comm_ref[d], wo_blk,
                                    preferred_element_type=jnp.float32)
        for d in (1, 2, 3):
            for b in range(B):
                rdmas[(d, b)].wait_send()

        out_ref[...] = out2d.astype(jnp.bfloat16).reshape(B, SQ, D_MODEL)

    return pl.pallas_call(
        body,
        out_shape=jax.ShapeDtypeStruct((B, SQ, D_MODEL), jnp.bfloat16),
        in_specs=[pl.BlockSpec(memory_space=pltpu.VMEM)] * 4,
        out_specs=pl.BlockSpec(memory_space=pltpu.VMEM),
        scratch_shapes=[
            pltpu.VMEM((N_DEV, ROWS, CHUNK), jnp.bfloat16),
            pltpu.SemaphoreType.DMA((N_DEV - 1, B)),
            pltpu.SemaphoreType.DMA((N_DEV - 1, B)),
        ],
        compiler_params=pltpu.CompilerParams(collective_id=0),
    )(x_bf, Wq_loc, KV, Wo)


# device time: 11292 ns/iter; 1.1985x vs baseline; 1.0704x over previous
import jax
import jax.numpy as jnp
from jax import lax
from jax.experimental import pallas as pl
from jax.experimental.pallas import tpu as pltpu

N_DEV = 4
B, SQ, SKV = 2, 128, 128
HQ_LOCAL, DH = 4, 64
D_MODEL = 512
CHUNK = HQ_LOCAL * DH
ROWS = B * SQ


def kernel(x, Wq, K_ext, V_ext, Wo):
    my_pos = lax.axis_index("i")
    Wq_loc = lax.dynamic_slice_in_dim(Wq, my_pos * CHUNK, CHUNK, axis=1)
    q_all = jnp.dot(x.reshape(ROWS, D_MODEL), Wq_loc,
                    preferred_element_type=jnp.float32).astype(jnp.bfloat16)
    KV = jnp.stack([
        jnp.transpose(K_ext, (0, 2, 3, 1)),
        jnp.transpose(V_ext, (0, 2, 3, 1)),
    ]).astype(jnp.bfloat16)

    def body(q_ref, kv_ref, wo_ref, out_ref,
             comm_ref, send_sems, recv_sems):
        me = lax.axis_index("i")

        barrier_sem = pltpu.get_barrier_semaphore()
        for d in range(1, N_DEV):
            pl.semaphore_signal(
                barrier_sem, inc=1,
                device_id=((me + d) % N_DEV,),
                device_id_type=pl.DeviceIdType.MESH,
            )

        q_all = q_ref[...]

        rdmas = {}
        ii = lax.broadcasted_iota(jnp.int32, (SQ, SKV), 0)
        jj = lax.broadcasted_iota(jnp.int32, (SQ, SKV), 1)
        mask = (jj // 64) <= (ii // 64)

        for b in range(B):
            for h in range(HQ_LOCAL):
                qh = q_all[b * SQ:(b + 1) * SQ, h * DH:(h + 1) * DH]
                khT = kv_ref[0, b, h]
                s = jnp.dot(qh, khT,
                            preferred_element_type=jnp.float32) * 0.125
                s = jnp.where(mask, s, -1e9)
                w = jnp.exp(s)
                w = (w / jnp.sum(w, axis=1, keepdims=True)).astype(jnp.bfloat16)
                ctx_h = lax.dot_general(
                    w, kv_ref[1, b, h], (((1,), (1,)), ((), ())),
                    preferred_element_type=jnp.float32,
                )
                comm_ref[0, b * SQ:(b + 1) * SQ, h * DH:(h + 1) * DH] = (
                    ctx_h.astype(jnp.bfloat16))

            if b == 0:
                pl.semaphore_wait(barrier_sem, N_DEV - 1)
            rows = pl.ds(b * SQ, SQ)
            for d in (2, 1, 3):
                rdmas[(d, b)] = pltpu.make_async_remote_copy(
                    src_ref=comm_ref.at[0, rows],
                    dst_ref=comm_ref.at[d, rows],
                    send_sem=send_sems.at[d - 1, b],
                    recv_sem=recv_sems.at[d - 1, b],
                    device_id=((me + d) % N_DEV,),
                    device_id_type=pl.DeviceIdType.MESH,
                )
                rdmas[(d, b)].start()

        wo_mine = wo_ref[pl.ds(me * CHUNK, CHUNK), :].astype(jnp.bfloat16)
        out2d = jnp.dot(comm_ref[0], wo_mine,
                        preferred_element_type=jnp.float32)

        for d in (1, 3, 2):
            origin = (me - d) % N_DEV
            wo_blk = wo_ref[pl.ds(origin * CHUNK, CHUNK), :].astype(
                jnp.bfloat16)
            for b in range(B):
                rdmas[(d, b)].wait_recv()
            out2d = out2d + jnp.dot(comm_ref[d], wo_blk,
                                    preferred_element_type=jnp.float32)
        for d in (1, 2, 3):
            for b in range(B):
                rdmas[(d, b)].wait_send()

        out_ref[...] = out2d.astype(jnp.bfloat16).reshape(B, SQ, D_MODEL)

    return pl.pallas_call(
        body,
        out_shape=jax.ShapeDtypeStruct((B, SQ, D_MODEL), jnp.bfloat16),
        in_specs=[pl.BlockSpec(memory_space=pltpu.VMEM)] * 3,
        out_specs=pl.BlockSpec(memory_space=pltpu.VMEM),
        scratch_shapes=[
            pltpu.VMEM((N_DEV, ROWS, CHUNK), jnp.bfloat16),
            pltpu.SemaphoreType.DMA((N_DEV - 1, B)),
            pltpu.SemaphoreType.DMA((N_DEV - 1, B)),
        ],
        compiler_params=pltpu.CompilerParams(collective_id=0),
    )(q_all, KV, Wo)
